# Initial kernel scaffold; baseline (speedup 1.0000x reference)
import jax
import jax.numpy as jnp
from jax.experimental import pallas as pl


def kernel(type_ids, lexicon_embeds):
    raise NotImplementedError("write your pallas kernel here")


# speedup vs baseline: 1.7693x; 1.7693x over previous
import functools

import jax
import jax.numpy as jnp
from jax import lax
from jax.experimental import pallas as pl
from jax.experimental.pallas import tpu as pltpu
from jax.experimental.pallas import tpu_sc as plsc

EMBED = 128
NUM_WORKERS = 32
CHUNK = 128


def _make_emb_kernel(n_rows: int):
    rows_per_w = n_rows // NUM_WORKERS
    n_chunks = rows_per_w // CHUNK
    mesh = plsc.VectorSubcoreMesh(core_axis_name="c", subcore_axis_name="s")

    @functools.partial(
        pl.kernel,
        mesh=mesh,
        out_type=jax.ShapeDtypeStruct((n_rows, EMBED), jnp.float32),
        scratch_types=[
            pltpu.VMEM((CHUNK,), jnp.int32),
            pltpu.VMEM((CHUNK, EMBED), jnp.float32),
            pltpu.SemaphoreType.DMA,
        ],
    )
    def emb(ids_hbm, table_hbm, out_hbm, idx_v, rows_v, sem):
        wid = lax.axis_index("s") * 2 + lax.axis_index("c")
        base = wid * rows_per_w

        def body(g, carry):
            off = base + g * CHUNK
            pltpu.sync_copy(ids_hbm.at[pl.ds(off, CHUNK)], idx_v)
            pltpu.async_copy(table_hbm.at[idx_v], rows_v, sem).wait()
            pltpu.sync_copy(rows_v, out_hbm.at[pl.ds(off, CHUNK)])
            return carry

        lax.fori_loop(0, n_chunks, body, 0)

    return emb


def kernel(type_ids, lexicon_embeds):
    batch, seq = type_ids.shape
    n_rows = batch * seq
    ids = type_ids.reshape(n_rows).astype(jnp.int32)
    out = _make_emb_kernel(n_rows)(ids, lexicon_embeds)
    return out.reshape(batch, seq, EMBED)


# speedup vs baseline: 1.7850x; 1.0089x over previous
import functools

import jax
import jax.numpy as jnp
from jax import lax
from jax.experimental import pallas as pl
from jax.experimental.pallas import tpu as pltpu
from jax.experimental.pallas import tpu_sc as plsc

EMBED = 128
NUM_WORKERS = 32
CHUNK = 128


def _make_emb_kernel(n_rows: int):
    rows_per_w = n_rows // NUM_WORKERS
    n_chunks = rows_per_w // CHUNK
    n_pairs = n_chunks // 2
    mesh = plsc.VectorSubcoreMesh(core_axis_name="c", subcore_axis_name="s")

    @functools.partial(
        pl.kernel,
        mesh=mesh,
        out_type=jax.ShapeDtypeStruct((n_rows, EMBED), jnp.float32),
        scratch_types=[
            pltpu.VMEM((n_chunks, CHUNK), jnp.int32),
            pltpu.VMEM((CHUNK, EMBED), jnp.float32),
            pltpu.VMEM((CHUNK, EMBED), jnp.float32),
            pltpu.SemaphoreType.DMA,
            pltpu.SemaphoreType.DMA,
            pltpu.SemaphoreType.DMA,
            pltpu.SemaphoreType.DMA,
        ],
    )
    def emb(ids_hbm, table_hbm, out_hbm, ids_v, rows0, rows1,
            gsem0, gsem1, osem0, osem1):
        wid = lax.axis_index("s") * 2 + lax.axis_index("c")
        base = wid * rows_per_w

        def gather(c, rows, sem):
            return pltpu.async_copy(table_hbm.at[ids_v.at[c]], rows, sem)

        def gather_wait(c, rows, sem):
            pltpu.make_async_copy(table_hbm.at[ids_v.at[c]], rows, sem).wait()

        def out_start(c, rows, sem):
            return pltpu.async_copy(
                rows, out_hbm.at[pl.ds(base + c * CHUNK, CHUNK)], sem)

        def out_wait(c, rows, sem):
            pltpu.make_async_copy(
                rows, out_hbm.at[pl.ds(base + c * CHUNK, CHUNK)], sem).wait()

        pltpu.sync_copy(ids_hbm.at[wid], ids_v)
        gather(0, rows0, gsem0)

        def body(j, carry):
            c0 = 2 * j
            c1 = c0 + 1
            gather_wait(c0, rows0, gsem0)

            @pl.when(j > 0)
            def _():
                out_wait(c1 - 2, rows1, osem1)

            gather(c1, rows1, gsem1)
            out_start(c0, rows0, osem0)
            gather_wait(c1, rows1, gsem1)
            out_wait(c0, rows0, osem0)

            @pl.when(j < n_pairs - 1)
            def _():
                gather(c0 + 2, rows0, gsem0)

            out_start(c1, rows1, osem1)
            return carry

        lax.fori_loop(0, n_pairs, body, 0)
        out_wait(n_chunks - 1, rows1, osem1)

    return emb


def kernel(type_ids, lexicon_embeds):
    batch, seq = type_ids.shape
    n_rows = batch * seq
    rows_per_w = n_rows // NUM_WORKERS
    n_chunks = rows_per_w // CHUNK
    ids = type_ids.reshape(NUM_WORKERS, n_chunks, CHUNK).astype(jnp.int32)
    out = _make_emb_kernel(n_rows)(ids, lexicon_embeds)
    return out.reshape(batch, seq, EMBED)
